# Initial kernel scaffold; baseline (speedup 1.0000x reference)
import jax
import jax.numpy as jnp
from jax.experimental import pallas as pl


def kernel(x, edge_index, W_msg1, b_msg1, W_next1, b_next1, W_msg2, b_msg2, W_next2, b_next2):
    raise NotImplementedError("write your pallas kernel here")


# speedup vs baseline: 3.0884x; 3.0884x over previous
import functools

import jax
import jax.numpy as jnp
from jax import lax
from jax.experimental import pallas as pl
from jax.experimental.pallas import tpu as pltpu
from jax.experimental.pallas import tpu_sc as plsc

N = 10000
E = 320000
D = 128
H = 256
HH = H // 2

NC = 2
NS = 16
NP = 10240
RT = 512
CHUNK = 128
NCHUNK = -(-E // (NS * CHUNK))
EPT = NCHUNK * CHUNK
EPAD = EPT * NS
RPT = NP // NS

_HIGHEST = lax.Precision.HIGHEST


def _dot(a, b):
    return jnp.dot(a, b, precision=_HIGHEST, preferred_element_type=jnp.float32)


def _tc1_body(x_ref, wd_ref, ws_ref, bm_ref, alo, ahi, blo, bhi):
    xt = x_ref[...]
    a = _dot(xt, wd_ref[...]) + bm_ref[...]
    b = _dot(xt, ws_ref[...])
    alo[...] = a[:, :HH]
    ahi[...] = a[:, HH:]
    blo[...] = b[:, :HH]
    bhi[...] = b[:, HH:]


def _tc2_body(x_ref, plo_ref, phi_ref, wa_ref, wblo_ref, wbhi_ref, bn_ref,
              wd2_ref, ws2_ref, bm2_ref, h2, alo, ahi, blo, bhi):
    h2t = (_dot(x_ref[...], wa_ref[...])
           + _dot(plo_ref[...], wblo_ref[...])
           + _dot(phi_ref[...], wbhi_ref[...])
           + bn_ref[...])
    h2[...] = h2t
    a2 = _dot(h2t, wd2_ref[...]) + bm2_ref[...]
    b2 = _dot(h2t, ws2_ref[...])
    alo[...] = a2[:, :HH]
    ahi[...] = a2[:, HH:]
    blo[...] = b2[:, :HH]
    bhi[...] = b2[:, HH:]


def _tc3_body(h2_ref, plo_ref, phi_ref, wa_ref, wblo_ref, wbhi_ref, bn_ref,
              out_ref):
    out_ref[...] = (_dot(h2_ref[...], wa_ref[...])
                    + _dot(plo_ref[...], wblo_ref[...])
                    + _dot(phi_ref[...], wbhi_ref[...])
                    + bn_ref[...])


def _row_spec(w):
    return pl.BlockSpec((RT, w), lambda i: (i, 0))


def _rep_spec(shape):
    return pl.BlockSpec(shape, lambda i: (0,) * len(shape))


_GRID = (NP // RT,)

_tc1 = pl.pallas_call(
    _tc1_body,
    grid=_GRID,
    in_specs=[_row_spec(D), _rep_spec((D, H)), _rep_spec((D, H)),
              _rep_spec((1, H))],
    out_specs=[_row_spec(HH)] * 4,
    out_shape=[jax.ShapeDtypeStruct((NP, HH), jnp.float32)] * 4,
)

_tc2 = pl.pallas_call(
    _tc2_body,
    grid=_GRID,
    in_specs=[_row_spec(D), _row_spec(HH), _row_spec(HH),
              _rep_spec((D, H)), _rep_spec((HH, H)), _rep_spec((HH, H)),
              _rep_spec((1, H)),
              _rep_spec((H, H)), _rep_spec((H, H)), _rep_spec((1, H))],
    out_specs=[_row_spec(H)] + [_row_spec(HH)] * 4,
    out_shape=([jax.ShapeDtypeStruct((NP, H), jnp.float32)]
               + [jax.ShapeDtypeStruct((NP, HH), jnp.float32)] * 4),
)

_tc3 = pl.pallas_call(
    _tc3_body,
    grid=_GRID,
    in_specs=[_row_spec(H), _row_spec(HH), _row_spec(HH),
              _rep_spec((H, H)), _rep_spec((HH, H)), _rep_spec((HH, H)),
              _rep_spec((1, H))],
    out_specs=_row_spec(H),
    out_shape=jax.ShapeDtypeStruct((NP, H), jnp.float32),
)


def _sc_edge_body(a_tab, b_tab, src_hbm, dst_hbm, p_out,
                  dst_v, ga_v, gb_v, src_v, abuf, bbuf, pooled_sh,
                  sem_a, sem_b):
    c = lax.axis_index("c")
    s = lax.axis_index("s")
    zvec = jnp.zeros((16,), jnp.float32)

    def _zrow(i, carry):
        for k in range(8):
            abuf[i, pl.ds(k * 16, 16)] = zvec
        return carry

    lax.fori_loop(0, CHUNK, _zrow, 0)
    for r in range(RPT // CHUNK):
        pltpu.sync_copy(abuf, pooled_sh.at[pl.ds(s * RPT + r * CHUNK, CHUNK)])
    plsc.subcore_barrier()

    goff = c * NP

    def _chunk(j, carry):
        base = s * EPT + j * CHUNK
        pltpu.sync_copy(dst_hbm.at[pl.ds(base, CHUNK)], dst_v)
        pltpu.sync_copy(src_hbm.at[pl.ds(base, CHUNK)], src_v)
        for k in range(8):
            sl = pl.ds(k * 16, 16)
            gb_v[sl] = src_v[sl] + goff
            ga_v[sl] = dst_v[sl] + goff
        db = pltpu.async_copy(b_tab.at[gb_v], bbuf, sem_b)
        da = pltpu.async_copy(a_tab.at[ga_v], abuf, sem_a)
        db.wait()
        da.wait()

        def _row(i, cc):
            for k in range(8):
                sl = pl.ds(k * 16, 16)
                abuf[i, sl] = jnp.maximum(abuf[i, sl] + bbuf[i, sl], 0.0)
            return cc

        lax.fori_loop(0, CHUNK, _row, 0)
        pltpu.sync_copy(abuf, pooled_sh.at[dst_v], add=True)
        return carry

    lax.fori_loop(0, NCHUNK, _chunk, 0)
    plsc.subcore_barrier()
    rs = pl.ds(s * RPT, RPT)
    pltpu.sync_copy(pooled_sh.at[rs], p_out.at[pl.ds(goff + s * RPT, RPT)])


@functools.cache
def _get_sc_edge():
  return pl.kernel(
    _sc_edge_body,
    out_type=jax.ShapeDtypeStruct((NC * NP, HH), jnp.float32),
    mesh=plsc.VectorSubcoreMesh(core_axis_name="c", subcore_axis_name="s"),
    scratch_types=[
        pltpu.VMEM((CHUNK,), jnp.int32),
        pltpu.VMEM((CHUNK,), jnp.int32),
        pltpu.VMEM((CHUNK,), jnp.int32),
        pltpu.VMEM((CHUNK,), jnp.int32),
        pltpu.VMEM((CHUNK, HH), jnp.float32),
        pltpu.VMEM((CHUNK, HH), jnp.float32),
        pltpu.VMEM_SHARED((NP, HH), jnp.float32),
        pltpu.SemaphoreType.DMA,
        pltpu.SemaphoreType.DMA,
    ],
  )


@jax.jit
def kernel(x, edge_index, W_msg1, b_msg1, W_next1, b_next1,
           W_msg2, b_msg2, W_next2, b_next2):
    src = edge_index[0]
    dst = edge_index[1]
    x_pad = jnp.zeros((NP, D), jnp.float32).at[:N].set(x)
    pad = EPAD - E
    src_p = jnp.concatenate([src, jnp.zeros((pad,), jnp.int32)])
    dst_p = jnp.concatenate([dst, jnp.full((pad,), N, jnp.int32)])

    alo, ahi, blo, bhi = _tc1(x_pad, W_msg1[:D], W_msg1[D:],
                              b_msg1.reshape(1, H))
    a_cat = jnp.concatenate([alo, ahi], axis=0)
    b_cat = jnp.concatenate([blo, bhi], axis=0)
    p1 = _get_sc_edge()(a_cat, b_cat, src_p, dst_p)

    h2, a2lo, a2hi, b2lo, b2hi = _tc2(
        x_pad, p1[:NP], p1[NP:],
        W_next1[:D], W_next1[D:D + HH], W_next1[D + HH:],
        b_next1.reshape(1, H),
        W_msg2[:H], W_msg2[H:], b_msg2.reshape(1, H))
    a2_cat = jnp.concatenate([a2lo, a2hi], axis=0)
    b2_cat = jnp.concatenate([b2lo, b2hi], axis=0)
    p2 = _get_sc_edge()(a2_cat, b2_cat, src_p, dst_p)

    out = _tc3(h2, p2[:NP], p2[NP:],
               W_next2[:H], W_next2[H:H + HH], W_next2[H + HH:],
               b_next2.reshape(1, H))
    return out[:N]


# speedup vs baseline: 3.1613x; 1.0236x over previous
import functools

import jax
import jax.numpy as jnp
from jax import lax
from jax.experimental import pallas as pl
from jax.experimental.pallas import tpu as pltpu
from jax.experimental.pallas import tpu_sc as plsc

N = 10000
E = 320000
D = 128
H = 256
HH = H // 2

NC = 2
NS = 16
NP = 10240
RT = 512
CHUNK = 64
IBC = 8
NCHUNK = 320
NIB = NCHUNK // IBC
EPT = NCHUNK * CHUNK
EPAD = EPT * NS
RPT = NP // NS

_HIGHEST = lax.Precision.HIGHEST


def _dot(a, b):
    return jnp.dot(a, b, precision=_HIGHEST, preferred_element_type=jnp.float32)


def _tc1_body(x_ref, wd_ref, ws_ref, bm_ref, alo, ahi, blo, bhi):
    xt = x_ref[...]
    a = _dot(xt, wd_ref[...]) + bm_ref[...]
    b = _dot(xt, ws_ref[...])
    alo[...] = a[:, :HH]
    ahi[...] = a[:, HH:]
    blo[...] = b[:, :HH]
    bhi[...] = b[:, HH:]


def _tc2_body(x_ref, plo_ref, phi_ref, wa_ref, wblo_ref, wbhi_ref, bn_ref,
              wd2_ref, ws2_ref, bm2_ref, h2, alo, ahi, blo, bhi):
    h2t = (_dot(x_ref[...], wa_ref[...])
           + _dot(plo_ref[...], wblo_ref[...])
           + _dot(phi_ref[...], wbhi_ref[...])
           + bn_ref[...])
    h2[...] = h2t
    a2 = _dot(h2t, wd2_ref[...]) + bm2_ref[...]
    b2 = _dot(h2t, ws2_ref[...])
    alo[...] = a2[:, :HH]
    ahi[...] = a2[:, HH:]
    blo[...] = b2[:, :HH]
    bhi[...] = b2[:, HH:]


def _tc3_body(h2_ref, plo_ref, phi_ref, wa_ref, wblo_ref, wbhi_ref, bn_ref,
              out_ref):
    out_ref[...] = (_dot(h2_ref[...], wa_ref[...])
                    + _dot(plo_ref[...], wblo_ref[...])
                    + _dot(phi_ref[...], wbhi_ref[...])
                    + bn_ref[...])


def _row_spec(w):
    return pl.BlockSpec((RT, w), lambda i: (i, 0))


def _rep_spec(shape):
    return pl.BlockSpec(shape, lambda i: (0,) * len(shape))


_GRID = (NP // RT,)

_tc1 = pl.pallas_call(
    _tc1_body,
    grid=_GRID,
    in_specs=[_row_spec(D), _rep_spec((D, H)), _rep_spec((D, H)),
              _rep_spec((1, H))],
    out_specs=[_row_spec(HH)] * 4,
    out_shape=[jax.ShapeDtypeStruct((NP, HH), jnp.float32)] * 4,
)

_tc2 = pl.pallas_call(
    _tc2_body,
    grid=_GRID,
    in_specs=[_row_spec(D), _row_spec(HH), _row_spec(HH),
              _rep_spec((D, H)), _rep_spec((HH, H)), _rep_spec((HH, H)),
              _rep_spec((1, H)),
              _rep_spec((H, H)), _rep_spec((H, H)), _rep_spec((1, H))],
    out_specs=[_row_spec(H)] + [_row_spec(HH)] * 4,
    out_shape=([jax.ShapeDtypeStruct((NP, H), jnp.float32)]
               + [jax.ShapeDtypeStruct((NP, HH), jnp.float32)] * 4),
)

_tc3 = pl.pallas_call(
    _tc3_body,
    grid=_GRID,
    in_specs=[_row_spec(H), _row_spec(HH), _row_spec(HH),
              _rep_spec((H, H)), _rep_spec((HH, H)), _rep_spec((HH, H)),
              _rep_spec((1, H))],
    out_specs=_row_spec(H),
    out_shape=jax.ShapeDtypeStruct((NP, H), jnp.float32),
)


def _sc_edge_body(a_tab, b_tab, srci_hbm, dsti_hbm, p_out,
                  dsti0, dsti1, srci0, srci1, ga0, ga1, gb0, gb1,
                  abuf0, abuf1, bbuf0, bbuf1, mbuf, pooled_sh,
                  sa0, sa1, sb0, sb1, ssc):
    c = lax.axis_index("c")
    s = lax.axis_index("s")
    dsti = (dsti0, dsti1)
    srci = (srci0, srci1)
    ga = (ga0, ga1)
    gb = (gb0, gb1)
    abuf = (abuf0, abuf1)
    bbuf = (bbuf0, bbuf1)
    sa = (sa0, sa1)
    sb = (sb0, sb1)
    goff = c * NP
    zvec = jnp.zeros((16,), jnp.float32)

    def load_iblock(ib, slot):
        row = s * NCHUNK + ib * IBC
        pltpu.sync_copy(dsti_hbm.at[pl.ds(row, IBC)], dsti[slot])
        pltpu.sync_copy(srci_hbm.at[pl.ds(row, IBC)], srci[slot])

    def issue_gather(slot, j, b):
        for k in range(CHUNK // 16):
            sl = pl.ds(k * 16, 16)
            ga[b][sl] = dsti[slot][j, sl] + goff
            gb[b][sl] = srci[slot][j, sl] + goff
        pltpu.async_copy(a_tab.at[ga[b]], abuf[b], sa[b])
        pltpu.async_copy(b_tab.at[gb[b]], bbuf[b], sb[b])

    def wait_gather(b):
        pltpu.make_async_copy(a_tab.at[ga[b]], abuf[b], sa[b]).wait()
        pltpu.make_async_copy(b_tab.at[gb[b]], bbuf[b], sb[b]).wait()

    def issue_scatter(slot, j):
        pltpu.async_copy(mbuf, pooled_sh.at[dsti[slot].at[j]], ssc, add=True)

    def wait_scatter(slot):
        pltpu.make_async_copy(mbuf, pooled_sh.at[dsti[slot].at[0]],
                              ssc).wait()

    def compute(b):
        def _row(i, carry):
            for k in range(HH // 16):
                sl = pl.ds(k * 16, 16)
                mbuf[i, sl] = jnp.maximum(abuf[b][i, sl] + bbuf[b][i, sl],
                                          0.0)
            return carry
        lax.fori_loop(0, CHUNK, _row, 0)

    def _zrow(i, carry):
        for k in range(HH // 16):
            abuf0[i, pl.ds(k * 16, 16)] = zvec
        return carry

    lax.fori_loop(0, CHUNK, _zrow, 0)
    for r in range(RPT // CHUNK):
        pltpu.sync_copy(abuf0,
                        pooled_sh.at[pl.ds(s * RPT + r * CHUNK, CHUNK)])

    load_iblock(0, 0)
    issue_gather(0, 0, 0)
    issue_gather(0, 1, 1)
    plsc.subcore_barrier()

    def block(ib, slot):
        nxt = 1 - slot

        @pl.when(ib + 1 < NIB)
        def _():
            load_iblock(ib + 1, nxt)

        def pair(g, carry):
            for b in (0, 1):
                j = 2 * g + b
                wait_gather(b)
                if b == 0:
                    @pl.when((g > 0) | (ib > 0))
                    def _():
                        wait_scatter(slot)
                else:
                    wait_scatter(slot)
                compute(b)
                issue_scatter(slot, j)

                @pl.when(g < IBC // 2 - 1)
                def _():
                    issue_gather(slot, j + 2, b)
            return carry

        lax.fori_loop(0, IBC // 2, pair, 0)

        @pl.when(ib + 1 < NIB)
        def _():
            issue_gather(nxt, 0, 0)
            issue_gather(nxt, 1, 1)

    def two_blocks(g2, carry):
        block(2 * g2, 0)
        block(2 * g2 + 1, 1)
        return carry

    lax.fori_loop(0, NIB // 2, two_blocks, 0)

    wait_scatter((NIB - 1) & 1)
    plsc.subcore_barrier()
    rs = pl.ds(s * RPT, RPT)
    pltpu.sync_copy(pooled_sh.at[rs], p_out.at[pl.ds(goff + s * RPT, RPT)])


@functools.cache
def _get_sc_edge():
  return pl.kernel(
    _sc_edge_body,
    out_type=jax.ShapeDtypeStruct((NC * NP, HH), jnp.float32),
    mesh=plsc.VectorSubcoreMesh(core_axis_name="c", subcore_axis_name="s"),
    scratch_types=(
        [pltpu.VMEM((IBC, CHUNK), jnp.int32)] * 4
        + [pltpu.VMEM((CHUNK,), jnp.int32)] * 4
        + [pltpu.VMEM((CHUNK, HH), jnp.float32)] * 5
        + [pltpu.VMEM_SHARED((NP, HH), jnp.float32)]
        + [pltpu.SemaphoreType.DMA] * 5
    ),
  )


@jax.jit
def kernel(x, edge_index, W_msg1, b_msg1, W_next1, b_next1,
           W_msg2, b_msg2, W_next2, b_next2):
    src = edge_index[0]
    dst = edge_index[1]
    x_pad = jnp.zeros((NP, D), jnp.float32).at[:N].set(x)
    pad = EPAD - E
    src_p = jnp.concatenate([src, jnp.zeros((pad,), jnp.int32)])
    dst_p = jnp.concatenate([dst, jnp.full((pad,), N, jnp.int32)])
    src_p = src_p.reshape(NS * NCHUNK, CHUNK)
    dst_p = dst_p.reshape(NS * NCHUNK, CHUNK)

    alo, ahi, blo, bhi = _tc1(x_pad, W_msg1[:D], W_msg1[D:],
                              b_msg1.reshape(1, H))
    a_cat = jnp.concatenate([alo, ahi], axis=0)
    b_cat = jnp.concatenate([blo, bhi], axis=0)
    p1 = _get_sc_edge()(a_cat, b_cat, src_p, dst_p)

    h2, a2lo, a2hi, b2lo, b2hi = _tc2(
        x_pad, p1[:NP], p1[NP:],
        W_next1[:D], W_next1[D:D + HH], W_next1[D + HH:],
        b_next1.reshape(1, H),
        W_msg2[:H], W_msg2[H:], b_msg2.reshape(1, H))
    a2_cat = jnp.concatenate([a2lo, a2hi], axis=0)
    b2_cat = jnp.concatenate([b2lo, b2hi], axis=0)
    p2 = _get_sc_edge()(a2_cat, b2_cat, src_p, dst_p)

    out = _tc3(h2, p2[:NP], p2[NP:],
               W_next2[:H], W_next2[H:H + HH], W_next2[H + HH:],
               b_next2.reshape(1, H))
    return out[:N]
